# baseline (speedup 1.0000x reference)
import functools

import jax
import jax.numpy as jnp
from jax import lax
from jax.experimental import pallas as pl
from jax.experimental.pallas import tpu as pltpu
from jax.experimental.pallas import tpu_sc as plsc

N = 10000
E = 320000
D = 128
NC = 2
NS = 16
NW = NC * NS
EPW = E // NW
K = 80
NCH = EPW // K
NP = 10240
RPT = NP // NS
DW = D

_mesh = plsc.VectorSubcoreMesh(core_axis_name="c", subcore_axis_name="s",
                               num_cores=NC, num_subcores=NS)


@functools.partial(
    pl.kernel,
    out_type=jax.ShapeDtypeStruct((NC, NP, DW), jnp.float32),
    mesh=_mesh,
    scratch_types=[
        pltpu.VMEM((NCH, K), jnp.int32),
        pltpu.VMEM((K, DW), jnp.float32),
        pltpu.VMEM_SHARED((NP, DW), jnp.float32),
    ],
)
def _deg_kernel(col_hbm, ones_hbm, zeros_hbm, degp_hbm, colv, onesv, acc):
    c = lax.axis_index("c")
    s = lax.axis_index("s")
    wid = s * NC + c
    base = s * RPT
    pltpu.sync_copy(col_hbm.at[wid], colv)
    pltpu.sync_copy(ones_hbm, onesv)
    pltpu.sync_copy(zeros_hbm.at[pl.ds(base, RPT)], acc.at[pl.ds(base, RPT)])
    plsc.subcore_barrier()

    def body(j, carry):
        pltpu.sync_copy(onesv, acc.at[colv.at[j]], add=True)
        return carry

    lax.fori_loop(0, NCH, body, 0)
    plsc.subcore_barrier()
    pltpu.sync_copy(acc.at[pl.ds(base, RPT)], degp_hbm.at[c, pl.ds(base, RPT)])


PK_SHIFT = 14


@functools.partial(
    pl.kernel,
    out_type=jax.ShapeDtypeStruct((NC, NP, D), jnp.float32),
    mesh=_mesh,
    scratch_types=[
        pltpu.VMEM((NCH, K), jnp.int32),
        pltpu.VMEM((4, K), jnp.int32),
        pltpu.VMEM((4, K), jnp.int32),
        pltpu.VMEM((K, D), jnp.float32),
        pltpu.VMEM((K, D), jnp.float32),
        pltpu.VMEM_SHARED((NP, D), jnp.float32),
        pltpu.SemaphoreType.DMA,
        pltpu.SemaphoreType.DMA,
        pltpu.SemaphoreType.DMA,
        pltpu.SemaphoreType.DMA,
    ],
)
def _msg_kernel(pk_hbm, h1_hbm, zeros_hbm, msgp_hbm,
                pkv, colb, rowb, buf0, buf1, acc, gsem0, gsem1, ssem0, ssem1):
    c = lax.axis_index("c")
    s = lax.axis_index("s")
    wid = s * NC + c
    base = s * RPT
    pltpu.sync_copy(pk_hbm.at[wid], pkv)
    pltpu.sync_copy(zeros_hbm.at[pl.ds(base, RPT)], acc.at[pl.ds(base, RPT)])

    def unpack(j, slot):
        for v in range(K // 16):
            pk = pkv[j, pl.ds(v * 16, 16)]
            colb[slot, pl.ds(v * 16, 16)] = jnp.bitwise_and(pk, (1 << PK_SHIFT) - 1)
            rowb[slot, pl.ds(v * 16, 16)] = jnp.right_shift(pk, PK_SHIFT)

    def gather(slot, buf, sem):
        pltpu.async_copy(h1_hbm.at[colb.at[slot]], buf, sem)

    def wait_gather(slot, buf, sem):
        pltpu.make_async_copy(h1_hbm.at[colb.at[slot]], buf, sem).wait()

    def scatter(slot, buf, sem):
        pltpu.async_copy(buf, acc.at[rowb.at[slot]], sem, add=True)

    def wait_scatter(slot, buf, sem):
        pltpu.make_async_copy(buf, acc.at[rowb.at[slot]], sem).wait()

    for j in range(4):
        unpack(j, j)
    plsc.subcore_barrier()

    gather(0, buf0, gsem0)
    gather(1, buf1, gsem1)

    def body(jo, carry):
        j0 = 4 * jo
        wait_gather(0, buf0, gsem0)
        scatter(0, buf0, ssem0)
        wait_gather(1, buf1, gsem1)
        scatter(1, buf1, ssem1)
        wait_scatter(0, buf0, ssem0)
        gather(2, buf0, gsem0)
        unpack(jnp.minimum(j0 + 4, NCH - 1), 0)
        wait_scatter(1, buf1, ssem1)
        gather(3, buf1, gsem1)
        unpack(jnp.minimum(j0 + 5, NCH - 1), 1)
        wait_gather(2, buf0, gsem0)
        scatter(2, buf0, ssem0)
        wait_gather(3, buf1, gsem1)
        scatter(3, buf1, ssem1)
        wait_scatter(2, buf0, ssem0)
        gather(0, buf0, gsem0)
        unpack(jnp.minimum(j0 + 6, NCH - 1), 2)
        wait_scatter(3, buf1, ssem1)
        gather(1, buf1, gsem1)
        unpack(jnp.minimum(j0 + 7, NCH - 1), 3)
        return carry

    lax.fori_loop(0, NCH // 4, body, 0)
    wait_gather(0, buf0, gsem0)
    scatter(0, buf0, ssem0)
    wait_gather(1, buf1, gsem1)
    wait_scatter(0, buf0, ssem0)
    plsc.subcore_barrier()
    pltpu.sync_copy(acc.at[pl.ds(base, RPT)], msgp_hbm.at[c, pl.ds(base, RPT)])


BN = 1000


def _h1_body(x_ref, w_ref, b_ref, degp_ref, h1_ref):
    dinv = lax.rsqrt(degp_ref[0] + degp_ref[1] + 1.0)
    h = lax.dot_general(x_ref[...], w_ref[...], (((1,), (1,)), ((), ())),
                        preferred_element_type=jnp.float32)
    h1_ref[...] = (h + b_ref[...]) * dinv


def _out_body(msgp_ref, h1_ref, degp_ref, o_ref):
    dinv = lax.rsqrt(degp_ref[0] + degp_ref[1] + 1.0)
    o_ref[...] = (msgp_ref[0] + msgp_ref[1] + h1_ref[...]) * dinv


def kernel(edge_index, X, W, b):
    col3 = edge_index[1].reshape(NW, NCH, K)
    pk3 = (jnp.left_shift(edge_index[0], PK_SHIFT) | edge_index[1]).reshape(
        NW, NCH, K)
    ones = jnp.ones((K, DW), jnp.float32)
    zm = jnp.zeros((NP, D), jnp.float32)
    zd = zm
    b2 = b.reshape(1, D)

    degp = _deg_kernel(col3, ones, zd)

    h1 = pl.pallas_call(
        _h1_body,
        grid=(N // BN,),
        in_specs=[
            pl.BlockSpec((BN, D), lambda i: (i, 0)),
            pl.BlockSpec((D, D), lambda i: (0, 0)),
            pl.BlockSpec((1, D), lambda i: (0, 0)),
            pl.BlockSpec((NC, BN, DW), lambda i: (0, i, 0)),
        ],
        out_specs=pl.BlockSpec((BN, D), lambda i: (i, 0)),
        out_shape=jax.ShapeDtypeStruct((N, D), jnp.float32),
    )(X, W, b2, degp)

    msgp = _msg_kernel(pk3, h1, zm)

    out = pl.pallas_call(
        _out_body,
        grid=(N // BN,),
        in_specs=[
            pl.BlockSpec((NC, BN, D), lambda i: (0, i, 0)),
            pl.BlockSpec((BN, D), lambda i: (i, 0)),
            pl.BlockSpec((NC, BN, DW), lambda i: (0, i, 0)),
        ],
        out_specs=pl.BlockSpec((BN, D), lambda i: (i, 0)),
        out_shape=jax.ShapeDtypeStruct((N, D), jnp.float32),
    )(msgp, h1, degp)

    return out


# speedup vs baseline: 1.1389x; 1.1389x over previous
import functools

import jax
import jax.numpy as jnp
from jax import lax
from jax.experimental import pallas as pl
from jax.experimental.pallas import tpu as pltpu
from jax.experimental.pallas import tpu_sc as plsc

N = 10000
E = 320000
D = 128
NC = 2
NS = 16
NW = NC * NS
EPW = E // NW
K = 80
NCH = EPW // K
NP = 10240
RPT = NP // NS
DW = D

_mesh = plsc.VectorSubcoreMesh(core_axis_name="c", subcore_axis_name="s",
                               num_cores=NC, num_subcores=NS)


@functools.partial(
    pl.kernel,
    out_type=jax.ShapeDtypeStruct((NC, NP, DW), jnp.float32),
    mesh=_mesh,
    scratch_types=[
        pltpu.VMEM((NCH, K), jnp.int32),
        pltpu.VMEM((K, DW), jnp.float32),
        pltpu.VMEM_SHARED((NP, DW), jnp.float32),
    ],
)
def _deg_kernel(col_hbm, ones_hbm, zeros_hbm, degp_hbm, colv, onesv, acc):
    c = lax.axis_index("c")
    s = lax.axis_index("s")
    wid = s * NC + c
    base = s * RPT
    pltpu.sync_copy(col_hbm.at[wid], colv)
    pltpu.sync_copy(ones_hbm, onesv)
    pltpu.sync_copy(zeros_hbm.at[pl.ds(base, RPT)], acc.at[pl.ds(base, RPT)])
    plsc.subcore_barrier()

    def body(j, carry):
        pltpu.sync_copy(onesv, acc.at[colv.at[j]], add=True)
        return carry

    lax.fori_loop(0, NCH, body, 0)
    plsc.subcore_barrier()
    pltpu.sync_copy(acc.at[pl.ds(base, RPT)], degp_hbm.at[c, pl.ds(base, RPT)])


PK_SHIFT = 14


@functools.partial(
    pl.kernel,
    out_type=jax.ShapeDtypeStruct((NC, NP, D), jnp.float32),
    mesh=_mesh,
    scratch_types=[
        pltpu.VMEM((NCH, K), jnp.int32),
        pltpu.VMEM((2, K), jnp.int32),
        pltpu.VMEM((2, K), jnp.int32),
        pltpu.VMEM((K, D), jnp.float32),
        pltpu.VMEM((K, D), jnp.float32),
        pltpu.VMEM_SHARED((NP, D), jnp.float32),
        pltpu.SemaphoreType.DMA,
        pltpu.SemaphoreType.DMA,
    ],
)
def _msg_kernel(pk_hbm, h1_hbm, zeros_hbm, msgp_hbm,
                pkv, colb, rowb, buf0, buf1, acc, sem0, sem1):
    c = lax.axis_index("c")
    s = lax.axis_index("s")
    wid = s * NC + c
    base = s * RPT
    pltpu.sync_copy(pk_hbm.at[wid], pkv)
    pltpu.sync_copy(zeros_hbm.at[pl.ds(base, RPT)], acc.at[pl.ds(base, RPT)])

    def unpack(j, slot):
        for v in range(K // 16):
            pk = pkv[j, pl.ds(v * 16, 16)]
            colb[slot, pl.ds(v * 16, 16)] = jnp.bitwise_and(pk, (1 << PK_SHIFT) - 1)
            rowb[slot, pl.ds(v * 16, 16)] = jnp.right_shift(pk, PK_SHIFT)

    unpack(0, 0)
    unpack(1, 1)
    plsc.subcore_barrier()

    pltpu.async_copy(h1_hbm.at[colb.at[0]], buf0, sem0)

    def body(jo, carry):
        j0 = 2 * jo
        pltpu.async_copy(h1_hbm.at[colb.at[1]], buf1, sem1)
        pltpu.make_async_copy(h1_hbm.at[colb.at[0]], buf0, sem0).wait()
        pltpu.sync_copy(buf0, acc.at[rowb.at[0]], add=True)
        unpack(jnp.minimum(j0 + 2, NCH - 1), 0)
        pltpu.async_copy(h1_hbm.at[colb.at[0]], buf0, sem0)
        pltpu.make_async_copy(h1_hbm.at[colb.at[1]], buf1, sem1).wait()
        pltpu.sync_copy(buf1, acc.at[rowb.at[1]], add=True)
        unpack(jnp.minimum(j0 + 3, NCH - 1), 1)
        return carry

    lax.fori_loop(0, (NCH - 1) // 2, body, 0)
    pltpu.make_async_copy(h1_hbm.at[colb.at[0]], buf0, sem0).wait()
    pltpu.sync_copy(buf0, acc.at[rowb.at[0]], add=True)
    plsc.subcore_barrier()
    pltpu.sync_copy(acc.at[pl.ds(base, RPT)], msgp_hbm.at[c, pl.ds(base, RPT)])


BN = 1000


def _h1_body(x_ref, w_ref, b_ref, degp_ref, h1_ref):
    dinv = lax.rsqrt(degp_ref[0] + degp_ref[1] + 1.0)
    h = lax.dot_general(x_ref[...], w_ref[...], (((1,), (1,)), ((), ())),
                        preferred_element_type=jnp.float32)
    h1_ref[...] = (h + b_ref[...]) * dinv


def _out_body(msgp_ref, h1_ref, degp_ref, o_ref):
    dinv = lax.rsqrt(degp_ref[0] + degp_ref[1] + 1.0)
    o_ref[...] = (msgp_ref[0] + msgp_ref[1] + h1_ref[...]) * dinv


def kernel(edge_index, X, W, b):
    col3 = edge_index[1].reshape(NW, NCH, K)
    pk3 = (jnp.left_shift(edge_index[0], PK_SHIFT) | edge_index[1]).reshape(
        NW, NCH, K)
    ones = jnp.ones((K, DW), jnp.float32)
    zm = jnp.zeros((NP, D), jnp.float32)
    zd = zm
    b2 = b.reshape(1, D)

    degp = _deg_kernel(col3, ones, zd)

    h1 = pl.pallas_call(
        _h1_body,
        grid=(N // BN,),
        in_specs=[
            pl.BlockSpec((BN, D), lambda i: (i, 0)),
            pl.BlockSpec((D, D), lambda i: (0, 0)),
            pl.BlockSpec((1, D), lambda i: (0, 0)),
            pl.BlockSpec((NC, BN, DW), lambda i: (0, i, 0)),
        ],
        out_specs=pl.BlockSpec((BN, D), lambda i: (i, 0)),
        out_shape=jax.ShapeDtypeStruct((N, D), jnp.float32),
    )(X, W, b2, degp)

    msgp = _msg_kernel(pk3, h1, zm)

    out = pl.pallas_call(
        _out_body,
        grid=(N // BN,),
        in_specs=[
            pl.BlockSpec((NC, BN, D), lambda i: (0, i, 0)),
            pl.BlockSpec((BN, D), lambda i: (i, 0)),
            pl.BlockSpec((NC, BN, DW), lambda i: (0, i, 0)),
        ],
        out_specs=pl.BlockSpec((BN, D), lambda i: (i, 0)),
        out_shape=jax.ShapeDtypeStruct((N, D), jnp.float32),
    )(msgp, h1, degp)

    return out


# speedup vs baseline: 1.1434x; 1.0040x over previous
import functools

import jax
import jax.numpy as jnp
from jax import lax
from jax.experimental import pallas as pl
from jax.experimental.pallas import tpu as pltpu
from jax.experimental.pallas import tpu_sc as plsc

N = 10000
E = 320000
D = 128
NC = 2
NS = 16
NW = NC * NS
EPW = E // NW
K = 80
NCH = EPW // K
NP = 10240
RPT = NP // NS
DW = D

_mesh = plsc.VectorSubcoreMesh(core_axis_name="c", subcore_axis_name="s",
                               num_cores=NC, num_subcores=NS)


@functools.partial(
    pl.kernel,
    out_type=jax.ShapeDtypeStruct((NC, NP, DW), jnp.float32),
    mesh=_mesh,
    scratch_types=[
        pltpu.VMEM((NCH, K), jnp.int32),
        pltpu.VMEM((K, DW), jnp.float32),
        pltpu.VMEM_SHARED((NP, DW), jnp.float32),
    ],
)
def _deg_kernel(col_hbm, ones_hbm, zeros_hbm, degp_hbm, colv, onesv, acc):
    c = lax.axis_index("c")
    s = lax.axis_index("s")
    wid = s * NC + c
    base = s * RPT
    pltpu.sync_copy(col_hbm.at[wid], colv)
    pltpu.sync_copy(ones_hbm, onesv)
    pltpu.sync_copy(zeros_hbm.at[pl.ds(base, RPT)], acc.at[pl.ds(base, RPT)])
    plsc.subcore_barrier()

    def body(j, carry):
        pltpu.sync_copy(onesv, acc.at[colv.at[j]], add=True)
        return carry

    lax.fori_loop(0, NCH, body, 0)
    plsc.subcore_barrier()
    pltpu.sync_copy(acc.at[pl.ds(base, RPT)], degp_hbm.at[c, pl.ds(base, RPT)])


PK_SHIFT = 14


@functools.partial(
    pl.kernel,
    out_type=jax.ShapeDtypeStruct((NC, NP, D), jnp.float32),
    mesh=_mesh,
    scratch_types=[
        pltpu.VMEM((NCH, K), jnp.int32),
        pltpu.VMEM((2, K), jnp.int32),
        pltpu.VMEM((2, K), jnp.int32),
        pltpu.VMEM((K, D), jnp.float32),
        pltpu.VMEM((K, D), jnp.float32),
        pltpu.VMEM_SHARED((NP, D), jnp.float32),
        pltpu.SemaphoreType.DMA,
        pltpu.SemaphoreType.DMA,
    ],
)
def _msg_kernel(pk_hbm, h1_hbm, zeros_hbm, msgp_hbm,
                pkv, colb, rowb, buf0, buf1, acc, sem0, sem1):
    c = lax.axis_index("c")
    s = lax.axis_index("s")
    wid = s * NC + c
    base = s * RPT
    pltpu.sync_copy(pk_hbm.at[wid], pkv)
    pltpu.sync_copy(zeros_hbm.at[pl.ds(base, RPT)], acc.at[pl.ds(base, RPT)])

    def unpack(j, slot):
        for v in range(K // 16):
            pk = pkv[j, pl.ds(v * 16, 16)]
            colb[slot, pl.ds(v * 16, 16)] = jnp.bitwise_and(pk, (1 << PK_SHIFT) - 1)
            rowb[slot, pl.ds(v * 16, 16)] = jnp.right_shift(pk, PK_SHIFT)

    unpack(0, 0)
    unpack(1, 1)
    plsc.subcore_barrier()

    pltpu.async_copy(h1_hbm.at[colb.at[0]], buf0, sem0)

    def body(jo, carry):
        j0 = 2 * jo
        pltpu.async_copy(h1_hbm.at[colb.at[1]], buf1, sem1)
        pltpu.make_async_copy(h1_hbm.at[colb.at[0]], buf0, sem0).wait()
        pltpu.sync_copy(buf0, acc.at[rowb.at[0]], add=True)
        unpack(jnp.minimum(j0 + 2, NCH - 1), 0)
        pltpu.async_copy(h1_hbm.at[colb.at[0]], buf0, sem0)
        pltpu.make_async_copy(h1_hbm.at[colb.at[1]], buf1, sem1).wait()
        pltpu.sync_copy(buf1, acc.at[rowb.at[1]], add=True)
        unpack(jnp.minimum(j0 + 3, NCH - 1), 1)
        return carry

    lax.fori_loop(0, (NCH - 1) // 2, body, 0)
    pltpu.make_async_copy(h1_hbm.at[colb.at[0]], buf0, sem0).wait()
    pltpu.sync_copy(buf0, acc.at[rowb.at[0]], add=True)
    plsc.subcore_barrier()
    pltpu.sync_copy(acc.at[pl.ds(base, RPT)], msgp_hbm.at[c, pl.ds(base, RPT)])


BN = 1000


def _mm_body(x_ref, w_ref, b_ref, h_ref):
    h = lax.dot_general(x_ref[...], w_ref[...], (((1,), (1,)), ((), ())),
                        preferred_element_type=jnp.float32)
    h_ref[...] = h + b_ref[...]


def _h1_body(h_ref, degp_ref, h1_ref):
    dinv = lax.rsqrt(degp_ref[0] + degp_ref[1] + 1.0)
    h1_ref[...] = h_ref[...] * dinv


def _out_body(msgp_ref, h1_ref, degp_ref, o_ref):
    dinv = lax.rsqrt(degp_ref[0] + degp_ref[1] + 1.0)
    o_ref[...] = (msgp_ref[0] + msgp_ref[1] + h1_ref[...]) * dinv


def kernel(edge_index, X, W, b):
    col3 = edge_index[1].reshape(NW, NCH, K)
    pk3 = (jnp.left_shift(edge_index[0], PK_SHIFT) | edge_index[1]).reshape(
        NW, NCH, K)
    ones = jnp.ones((K, DW), jnp.float32)
    zm = jnp.zeros((NP, D), jnp.float32)
    zd = zm
    b2 = b.reshape(1, D)

    h = pl.pallas_call(
        _mm_body,
        grid=(N // BN,),
        in_specs=[
            pl.BlockSpec((BN, D), lambda i: (i, 0)),
            pl.BlockSpec((D, D), lambda i: (0, 0)),
            pl.BlockSpec((1, D), lambda i: (0, 0)),
        ],
        out_specs=pl.BlockSpec((BN, D), lambda i: (i, 0)),
        out_shape=jax.ShapeDtypeStruct((N, D), jnp.float32),
    )(X, W, b2)

    degp = _deg_kernel(col3, ones, zd)

    h1 = pl.pallas_call(
        _h1_body,
        grid=(N // BN,),
        in_specs=[
            pl.BlockSpec((BN, D), lambda i: (i, 0)),
            pl.BlockSpec((NC, BN, DW), lambda i: (0, i, 0)),
        ],
        out_specs=pl.BlockSpec((BN, D), lambda i: (i, 0)),
        out_shape=jax.ShapeDtypeStruct((N, D), jnp.float32),
    )(h, degp)

    msgp = _msg_kernel(pk3, h1, zm)

    out = pl.pallas_call(
        _out_body,
        grid=(N // BN,),
        in_specs=[
            pl.BlockSpec((NC, BN, D), lambda i: (0, i, 0)),
            pl.BlockSpec((BN, D), lambda i: (i, 0)),
            pl.BlockSpec((NC, BN, DW), lambda i: (0, i, 0)),
        ],
        out_specs=pl.BlockSpec((BN, D), lambda i: (i, 0)),
        out_shape=jax.ShapeDtypeStruct((N, D), jnp.float32),
    )(msgp, h1, degp)

    return out


# speedup vs baseline: 1.1471x; 1.0033x over previous
import functools

import jax
import jax.numpy as jnp
from jax import lax
from jax.experimental import pallas as pl
from jax.experimental.pallas import tpu as pltpu
from jax.experimental.pallas import tpu_sc as plsc

N = 10000
E = 320000
D = 128
NC = 2
NS = 16
NW = NC * NS
EPW = E // NW
K = 80
NCH = EPW // K
NP = 10240
KD = 128
NCHD = 79
RPT = NP // NS
DW = D

_mesh = plsc.VectorSubcoreMesh(core_axis_name="c", subcore_axis_name="s",
                               num_cores=NC, num_subcores=NS)


@functools.partial(
    pl.kernel,
    out_type=jax.ShapeDtypeStruct((NC, NP, DW), jnp.float32),
    mesh=_mesh,
    scratch_types=[
        pltpu.VMEM((NCHD, KD), jnp.int32),
        pltpu.VMEM((KD, DW), jnp.float32),
        pltpu.VMEM_SHARED((NP, DW), jnp.float32),
    ],
)
def _deg_kernel(col_hbm, ones_hbm, zeros_hbm, degp_hbm, colv, onesv, acc):
    c = lax.axis_index("c")
    s = lax.axis_index("s")
    wid = s * NC + c
    base = s * RPT
    pltpu.sync_copy(col_hbm.at[wid], colv)
    pltpu.sync_copy(ones_hbm, onesv)
    pltpu.sync_copy(zeros_hbm.at[pl.ds(base, RPT)], acc.at[pl.ds(base, RPT)])
    plsc.subcore_barrier()

    def body(j, carry):
        pltpu.sync_copy(onesv, acc.at[colv.at[j]], add=True)
        return carry

    lax.fori_loop(0, NCHD, body, 0)
    plsc.subcore_barrier()
    pltpu.sync_copy(acc.at[pl.ds(base, RPT)], degp_hbm.at[c, pl.ds(base, RPT)])


PK_SHIFT = 14


@functools.partial(
    pl.kernel,
    out_type=jax.ShapeDtypeStruct((NC, NP, D), jnp.float32),
    mesh=_mesh,
    scratch_types=[
        pltpu.VMEM((NCH, K), jnp.int32),
        pltpu.VMEM((2, K), jnp.int32),
        pltpu.VMEM((2, K), jnp.int32),
        pltpu.VMEM((K, D), jnp.float32),
        pltpu.VMEM((K, D), jnp.float32),
        pltpu.VMEM_SHARED((NP, D), jnp.float32),
        pltpu.SemaphoreType.DMA,
        pltpu.SemaphoreType.DMA,
    ],
)
def _msg_kernel(pk_hbm, h1_hbm, zeros_hbm, msgp_hbm,
                pkv, colb, rowb, buf0, buf1, acc, sem0, sem1):
    c = lax.axis_index("c")
    s = lax.axis_index("s")
    wid = s * NC + c
    base = s * RPT
    pltpu.sync_copy(pk_hbm.at[wid], pkv)
    pltpu.sync_copy(zeros_hbm.at[pl.ds(base, RPT)], acc.at[pl.ds(base, RPT)])

    def unpack(j, slot):
        for v in range(K // 16):
            pk = pkv[j, pl.ds(v * 16, 16)]
            colb[slot, pl.ds(v * 16, 16)] = jnp.bitwise_and(pk, (1 << PK_SHIFT) - 1)
            rowb[slot, pl.ds(v * 16, 16)] = jnp.right_shift(pk, PK_SHIFT)

    unpack(0, 0)
    unpack(1, 1)
    plsc.subcore_barrier()

    pltpu.async_copy(h1_hbm.at[colb.at[0]], buf0, sem0)

    def body(jo, carry):
        j0 = 2 * jo
        pltpu.async_copy(h1_hbm.at[colb.at[1]], buf1, sem1)
        pltpu.make_async_copy(h1_hbm.at[colb.at[0]], buf0, sem0).wait()
        pltpu.sync_copy(buf0, acc.at[rowb.at[0]], add=True)
        unpack(jnp.minimum(j0 + 2, NCH - 1), 0)
        pltpu.async_copy(h1_hbm.at[colb.at[0]], buf0, sem0)
        pltpu.make_async_copy(h1_hbm.at[colb.at[1]], buf1, sem1).wait()
        pltpu.sync_copy(buf1, acc.at[rowb.at[1]], add=True)
        unpack(jnp.minimum(j0 + 3, NCH - 1), 1)
        return carry

    lax.fori_loop(0, (NCH - 1) // 2, body, 0)
    pltpu.make_async_copy(h1_hbm.at[colb.at[0]], buf0, sem0).wait()
    pltpu.sync_copy(buf0, acc.at[rowb.at[0]], add=True)
    plsc.subcore_barrier()
    pltpu.sync_copy(acc.at[pl.ds(base, RPT)], msgp_hbm.at[c, pl.ds(base, RPT)])


BN = 1000


def _mm_body(x_ref, w_ref, b_ref, h_ref):
    h = lax.dot_general(x_ref[...], w_ref[...], (((1,), (1,)), ((), ())),
                        preferred_element_type=jnp.float32)
    h_ref[...] = h + b_ref[...]


def _h1_body(h_ref, degp_ref, h1_ref):
    dinv = lax.rsqrt(degp_ref[0] + degp_ref[1] + 1.0)
    h1_ref[...] = h_ref[...] * dinv


def _out_body(msgp_ref, h1_ref, degp_ref, o_ref):
    dinv = lax.rsqrt(degp_ref[0] + degp_ref[1] + 1.0)
    o_ref[...] = (msgp_ref[0] + msgp_ref[1] + h1_ref[...]) * dinv


def kernel(edge_index, X, W, b):
    colw = edge_index[1].reshape(NW, EPW)
    pad = N + jnp.broadcast_to(
        jnp.arange(NCHD * KD - EPW, dtype=jnp.int32) % (NP - N), (NW, 112))
    col3 = jnp.concatenate([colw, pad], axis=1).reshape(NW, NCHD, KD)
    pk3 = (jnp.left_shift(edge_index[0], PK_SHIFT) | edge_index[1]).reshape(
        NW, NCH, K)
    ones = jnp.ones((KD, DW), jnp.float32)
    zm = jnp.zeros((NP, D), jnp.float32)
    zd = zm
    b2 = b.reshape(1, D)

    h = pl.pallas_call(
        _mm_body,
        grid=(N // BN,),
        in_specs=[
            pl.BlockSpec((BN, D), lambda i: (i, 0)),
            pl.BlockSpec((D, D), lambda i: (0, 0)),
            pl.BlockSpec((1, D), lambda i: (0, 0)),
        ],
        out_specs=pl.BlockSpec((BN, D), lambda i: (i, 0)),
        out_shape=jax.ShapeDtypeStruct((N, D), jnp.float32),
    )(X, W, b2)

    degp = _deg_kernel(col3, ones, zd)

    h1 = pl.pallas_call(
        _h1_body,
        grid=(N // BN,),
        in_specs=[
            pl.BlockSpec((BN, D), lambda i: (i, 0)),
            pl.BlockSpec((NC, BN, DW), lambda i: (0, i, 0)),
        ],
        out_specs=pl.BlockSpec((BN, D), lambda i: (i, 0)),
        out_shape=jax.ShapeDtypeStruct((N, D), jnp.float32),
    )(h, degp)

    msgp = _msg_kernel(pk3, h1, zm)

    out = pl.pallas_call(
        _out_body,
        grid=(N // BN,),
        in_specs=[
            pl.BlockSpec((NC, BN, D), lambda i: (0, i, 0)),
            pl.BlockSpec((BN, D), lambda i: (i, 0)),
            pl.BlockSpec((NC, BN, DW), lambda i: (0, i, 0)),
        ],
        out_specs=pl.BlockSpec((BN, D), lambda i: (i, 0)),
        out_shape=jax.ShapeDtypeStruct((N, D), jnp.float32),
    )(msgp, h1, degp)

    return out
